# initial kernel scaffold (unmeasured)
import jax
import jax.numpy as jnp
from jax import lax
from jax.experimental import pallas as pl
from jax.experimental.pallas import tpu as pltpu


def kernel(
    x,
):
    def body(*refs):
        pass

    out_shape = jax.ShapeDtypeStruct(..., jnp.float32)
    return pl.pallas_call(body, out_shape=out_shape)(...)


# baseline (device time: 104700 ns/iter reference)
import jax
import jax.numpy as jnp
from jax import lax
from jax.experimental import pallas as pl
from jax.experimental.pallas import tpu as pltpu


def kernel(x):
    m, n = x.shape

    def body(x_ref, out_ref, send_buf, recv_buf, send_sem, recv_sem):
        my_x = lax.axis_index("x")
        my_y = lax.axis_index("y")
        nbr = (my_x, 1 - my_y)

        barrier_sem = pltpu.get_barrier_semaphore()
        pl.semaphore_signal(
            barrier_sem, inc=1, device_id=nbr,
            device_id_type=pl.DeviceIdType.MESH,
        )
        pl.semaphore_wait(barrier_sem, 1)

        send_buf[...] = x_ref[...].astype(jnp.bfloat16)
        rdma = pltpu.make_async_remote_copy(
            src_ref=send_buf,
            dst_ref=recv_buf,
            send_sem=send_sem,
            recv_sem=recv_sem,
            device_id=nbr,
            device_id_type=pl.DeviceIdType.MESH,
        )
        rdma.start()
        rdma.wait()

        out_ref[...] = (
            x_ref[...] + recv_buf[...].astype(jnp.float32)
        ).astype(jnp.bfloat16)

    return pl.pallas_call(
        body,
        out_shape=jax.ShapeDtypeStruct((m, n), jnp.bfloat16),
        in_specs=[pl.BlockSpec(memory_space=pltpu.VMEM)],
        out_specs=pl.BlockSpec(memory_space=pltpu.VMEM),
        scratch_shapes=[
            pltpu.VMEM((m, n), jnp.bfloat16),
            pltpu.VMEM((m, n), jnp.bfloat16),
            pltpu.SemaphoreType.DMA,
            pltpu.SemaphoreType.DMA,
        ],
        compiler_params=pltpu.CompilerParams(collective_id=0),
    )(x)


# device time: 66011 ns/iter; 1.5861x vs baseline; 1.5861x over previous
import jax
import jax.numpy as jnp
from jax import lax
from jax.experimental import pallas as pl
from jax.experimental.pallas import tpu as pltpu

C = 8


def kernel(x):
    m, n = x.shape
    half = m // 2
    r = half // C

    def body(x_ref, out_ref, ysend, yrecv, ysend_sems, yrecv_sems,
             xsend_sems, xrecv_sems):
        my_x = lax.axis_index("x")
        my_y = lax.axis_index("y")
        nbr_y = (my_x, 1 - my_y)
        nbr_x = (1 - my_x, my_y)

        h0 = my_x * half

        barrier_sem = pltpu.get_barrier_semaphore()
        for nbr in (nbr_y, nbr_x):
            pl.semaphore_signal(
                barrier_sem, inc=1, device_id=nbr,
                device_id_type=pl.DeviceIdType.MESH,
            )
        pl.semaphore_wait(barrier_sem, 2)

        def rdma_a(c):
            return pltpu.make_async_remote_copy(
                src_ref=ysend.at[pl.ds(c * r, r)],
                dst_ref=yrecv.at[pl.ds(c * r, r)],
                send_sem=ysend_sems.at[c],
                recv_sem=yrecv_sems.at[c],
                device_id=nbr_y,
                device_id_type=pl.DeviceIdType.MESH,
            )

        def rdma_b(c):
            return pltpu.make_async_remote_copy(
                src_ref=out_ref.at[pl.ds(h0 + c * r, r)],
                dst_ref=out_ref.at[pl.ds(h0 + c * r, r)],
                send_sem=xsend_sems.at[c],
                recv_sem=xrecv_sems.at[c],
                device_id=nbr_x,
                device_id_type=pl.DeviceIdType.MESH,
            )

        for c in range(C):
            ysend[pl.ds(c * r, r)] = x_ref[pl.ds(h0 + c * r, r)].astype(
                jnp.bfloat16
            )
            rdma_a(c).start()

        for c in range(C):
            rdma_a(c).wait_recv()
            out_ref[pl.ds(h0 + c * r, r)] = (
                x_ref[pl.ds(h0 + c * r, r)]
                + yrecv[pl.ds(c * r, r)].astype(jnp.float32)
            ).astype(jnp.bfloat16)
            rdma_b(c).start()

        for c in range(C):
            rdma_b(c).wait_recv()
        for c in range(C):
            rdma_a(c).wait_send()
            rdma_b(c).wait_send()

    return pl.pallas_call(
        body,
        out_shape=jax.ShapeDtypeStruct((m, n), jnp.bfloat16),
        in_specs=[pl.BlockSpec(memory_space=pltpu.VMEM)],
        out_specs=pl.BlockSpec(memory_space=pltpu.VMEM),
        scratch_shapes=[
            pltpu.VMEM((half, n), jnp.bfloat16),
            pltpu.VMEM((half, n), jnp.bfloat16),
            pltpu.SemaphoreType.DMA((C,)),
            pltpu.SemaphoreType.DMA((C,)),
            pltpu.SemaphoreType.DMA((C,)),
            pltpu.SemaphoreType.DMA((C,)),
        ],
        compiler_params=pltpu.CompilerParams(collective_id=0),
    )(x)


# device time: 59200 ns/iter; 1.7686x vs baseline; 1.1151x over previous
import json as _json
import os as _os

import jax
import jax.numpy as jnp
from jax import lax
from jax.experimental import pallas as pl
from jax.experimental.pallas import tpu as pltpu

_cfg = {}
_cfg_path = _os.path.join(_os.path.dirname(__file__), "sck_cfg.json")
if _os.path.exists(_cfg_path):
    _cfg = _json.loads(open(_cfg_path).read())

C = int(_cfg.get("C", 16))
L = int(_cfg.get("L", 2))
D = int(_cfg.get("D", 3))


def _chunk_sizes(half):
    if "SIZES" in _cfg:
        sizes = [int(s) for s in _cfg["SIZES"]]
    elif half == 2048:
        sizes = [128] * 15 + [64, 64]
    else:
        sizes = [half // C] * C
    assert sum(sizes) == half, (sizes, half)
    return sizes


def kernel(x):
    m, n = x.shape
    half = m // 2
    sizes = _chunk_sizes(half)
    offs = [sum(sizes[:c]) for c in range(len(sizes))]
    nc = len(sizes)
    rmax = max(sizes)

    def body(x_ref, out_ref, xtmp, ysend, yrecv, hs, load_sems, store_sems,
             ysend_sems, yrecv_sems, xsend_sems, xrecv_sems):
        my_x = lax.axis_index("x")
        my_y = lax.axis_index("y")
        nbr_y = (my_x, 1 - my_y)
        nbr_x = (1 - my_x, my_y)

        h0 = my_x * half

        def load(c):
            return pltpu.make_async_copy(
                x_ref.at[pl.ds(h0 + offs[c], sizes[c])],
                xtmp.at[c % D, pl.ds(0, sizes[c])],
                load_sems.at[c % D],
            )

        def store(c):
            return pltpu.make_async_copy(
                hs.at[pl.ds(offs[c], sizes[c])],
                out_ref.at[pl.ds(h0 + offs[c], sizes[c])],
                store_sems.at[c],
            )

        for c in range(min(D, nc)):
            load(c).start()
        npre = min(2, nc)
        for c in range(npre):
            load(c).wait()
            ysend[pl.ds(offs[c], sizes[c])] = xtmp[
                c % D, pl.ds(0, sizes[c])
            ].astype(jnp.bfloat16)

        barrier_sem = pltpu.get_barrier_semaphore()
        for nbr in (nbr_y, nbr_x):
            pl.semaphore_signal(
                barrier_sem, inc=1, device_id=nbr,
                device_id_type=pl.DeviceIdType.MESH,
            )
        pl.semaphore_wait(barrier_sem, 2)

        def rdma_a(c):
            return pltpu.make_async_remote_copy(
                src_ref=ysend.at[pl.ds(offs[c], sizes[c])],
                dst_ref=yrecv.at[pl.ds(offs[c], sizes[c])],
                send_sem=ysend_sems.at[c],
                recv_sem=yrecv_sems.at[c],
                device_id=nbr_y,
                device_id_type=pl.DeviceIdType.MESH,
            )

        def rdma_b(c):
            return pltpu.make_async_remote_copy(
                src_ref=hs.at[pl.ds(offs[c], sizes[c])],
                dst_ref=out_ref.at[pl.ds(h0 + offs[c], sizes[c])],
                send_sem=xsend_sems.at[c],
                recv_sem=xrecv_sems.at[c],
                device_id=nbr_x,
                device_id_type=pl.DeviceIdType.MESH,
            )

        def reduce_and_forward(c):
            rdma_a(c).wait_recv()
            hs[pl.ds(offs[c], sizes[c])] = (
                ysend[pl.ds(offs[c], sizes[c])]
                + yrecv[pl.ds(offs[c], sizes[c])]
            )
            rdma_b(c).start()
            store(c).start()

        for c in range(nc):
            if c >= npre:
                load(c).wait()
                ysend[pl.ds(offs[c], sizes[c])] = xtmp[
                    c % D, pl.ds(0, sizes[c])
                ].astype(jnp.bfloat16)
            rdma_a(c).start()
            if c + D < nc:
                load(c + D).start()
            if c >= L:
                reduce_and_forward(c - L)
        for c in range(nc - L, nc):
            reduce_and_forward(c)

        for c in range(nc):
            rdma_b(c).wait_recv()
        for c in range(nc):
            store(c).wait()
            rdma_a(c).wait_send()
            rdma_b(c).wait_send()

    return pl.pallas_call(
        body,
        out_shape=jax.ShapeDtypeStruct((m, n), jnp.bfloat16),
        in_specs=[pl.BlockSpec(memory_space=pl.ANY)],
        out_specs=pl.BlockSpec(memory_space=pl.ANY),
        scratch_shapes=[
            pltpu.VMEM((D, rmax, n), jnp.float32),
            pltpu.VMEM((half, n), jnp.bfloat16),
            pltpu.VMEM((half, n), jnp.bfloat16),
            pltpu.VMEM((half, n), jnp.bfloat16),
            pltpu.SemaphoreType.DMA((D,)),
            pltpu.SemaphoreType.DMA((nc,)),
            pltpu.SemaphoreType.DMA((nc,)),
            pltpu.SemaphoreType.DMA((nc,)),
            pltpu.SemaphoreType.DMA((nc,)),
            pltpu.SemaphoreType.DMA((nc,)),
        ],
        compiler_params=pltpu.CompilerParams(collective_id=0),
    )(x)
